# Initial kernel scaffold; baseline (speedup 1.0000x reference)
import jax
import jax.numpy as jnp
from jax.experimental import pallas as pl


def kernel(grad, seg, fV, nV):
    raise NotImplementedError("write your pallas kernel here")


# speedup vs baseline: 13.6474x; 13.6474x over previous
import jax
import jax.numpy as jnp
from jax import lax
from jax.experimental import pallas as pl
from jax.experimental.pallas import tpu as pltpu
from jax.experimental.pallas import tpu_sc as plsc

P = 16
EPS = 1e-07
NSEG = 16384
PB = P * P
NBINS = NSEG * PB
NPIX = 8 * 512 * 512

CHUNK = 1 << 20
N_PASS = 2

N_SUBCORES = 16
PIX_PER_TILE = NPIX // N_SUBCORES
W = 16384
NW = PIX_PER_TILE // W


def _pos_body(grad_ref, seg_ref, pos_ref):
    g = grad_ref[0]
    seg = seg_ref[0]
    lo = EPS - 1.0
    hi = 1.0 - EPS
    gy = jnp.clip(g[0], lo, hi)
    gx = jnp.clip(g[1], lo, hi)
    yi = ((gy + 1.0) * (P / 2.0)).astype(jnp.int32)
    xi = ((gx + 1.0) * (P / 2.0)).astype(jnp.int32)
    pos_ref[0] = seg * PB + yi * P + xi


def _compute_pos(grad, seg):
    return pl.pallas_call(
        _pos_body,
        grid=(8,),
        in_specs=[
            pl.BlockSpec((1, 2, 512, 512), lambda i: (i, 0, 0, 0)),
            pl.BlockSpec((1, 512, 512), lambda i: (i, 0, 0)),
        ],
        out_specs=pl.BlockSpec((1, 512, 512), lambda i: (i, 0, 0)),
        out_shape=jax.ShapeDtypeStruct((8, 512, 512), jnp.int32),
    )(grad, seg)


def _hist_body(pos_hbm, hist_hbm, chunk_sh, idx_v, val_v, zero_v):
    c = lax.axis_index("c")
    s = lax.axis_index("s")

    zeros16 = jnp.zeros((16,), jnp.float32)

    @pl.loop(0, zero_v.shape[0] // 16)
    def _fill_zero(i):
        zero_v[pl.ds(i * 16, 16)] = zeros16

    zlen = zero_v.shape[0]
    slice_per_tile = CHUNK // N_SUBCORES
    pix_base = s * PIX_PER_TILE

    for p in range(N_PASS):
        base = (p * 2) * CHUNK + c * CHUNK

        @pl.loop(0, slice_per_tile // zlen)
        def _zero_chunk(j):
            pltpu.sync_copy(zero_v,
                            chunk_sh.at[pl.ds(s * slice_per_tile + j * zlen,
                                              zlen)])

        plsc.subcore_barrier()

        for w in range(NW):
            pltpu.sync_copy(pos_hbm.at[pl.ds(pix_base + w * W, W)], idx_v)

            @pl.loop(0, W // 16, unroll=4)
            def _remap(i):
                idx = idx_v[pl.ds(i * 16, 16)]
                local = idx - base
                ok = (local >= 0) & (local < CHUNK)
                spread = idx & (CHUNK - 1)
                idx_v[pl.ds(i * 16, 16)] = jnp.where(ok, local, spread)
                val_v[pl.ds(i * 16, 16)] = jnp.where(ok, 1.0, 0.0)

            pltpu.sync_copy(val_v, chunk_sh.at[idx_v], add=True)

        plsc.subcore_barrier()

        pltpu.sync_copy(chunk_sh.at[pl.ds(s * slice_per_tile, slice_per_tile)],
                        hist_hbm.at[pl.ds(base + s * slice_per_tile,
                                          slice_per_tile)])


def _scatter_hist(pos_flat):
    kern = pl.kernel(
        _hist_body,
        out_type=jax.ShapeDtypeStruct((NBINS,), jnp.float32),
        mesh=plsc.VectorSubcoreMesh(core_axis_name="c", subcore_axis_name="s"),
        compiler_params=pltpu.CompilerParams(needs_layout_passes=False),
        scratch_types=[
            pltpu.VMEM_SHARED((CHUNK,), jnp.float32),
            pltpu.VMEM((W,), jnp.int32),
            pltpu.VMEM((W,), jnp.float32),
            pltpu.VMEM((8192,), jnp.float32),
        ],
    )
    return kern(pos_flat)


def _final_body(hist_ref, out_ref):
    h = hist_ref[...]
    sizes = jnp.sum(h, axis=1, keepdims=True)
    out_ref[...] = h / (sizes * ((P / 32.0) ** 2))


def _finalize(hist2d):
    return pl.pallas_call(
        _final_body,
        grid=(8,),
        in_specs=[pl.BlockSpec((2048, PB), lambda i: (i, 0))],
        out_specs=pl.BlockSpec((2048, PB), lambda i: (i, 0)),
        out_shape=jax.ShapeDtypeStruct((NSEG, PB), jnp.float32),
    )(hist2d)


def kernel(grad, seg, fV, nV):
    pos = _compute_pos(grad, seg.astype(jnp.int32))
    hist = _scatter_hist(pos.reshape(NPIX))
    out = _finalize(hist.reshape(NSEG, PB))
    return out.reshape(NSEG, 1, P, P)
